# baseline (speedup 1.0000x reference)
import jax
import jax.numpy as jnp
from jax import lax
from jax.experimental import pallas as pl
from jax.experimental.pallas import tpu as pltpu
from jax.experimental.pallas import tpu_sc as plsc

N = 10000
NPAD = 10240
D = 128
NC = 2
NS = 16
NW = NC * NS
ROWS_PER_TILE = NPAD // NS
BATCH = 128
BM = 2560
GRID = NPAD // BM
F32 = jnp.float32


def _sc_deg_body(d_hbm, out_hbm, d_v, ones_v, z_v, deg_sh):
  c = lax.axis_index("c")
  s = lax.axis_index("s")
  w = c * NS + s
  nb = d_v.shape[0]

  pltpu.sync_copy(d_hbm.at[w], d_v)

  def mk_ones(i, carry):
    ones_v[pl.ds(i * 16, 16)] = jnp.full((16,), 1.0, F32)
    return carry
  lax.fori_loop(0, BATCH // 16, mk_ones, 0)

  def mk_zero(i, carry):
    z_v[pl.ds(i * 16, 16)] = jnp.zeros((16,), F32)
    return carry
  lax.fori_loop(0, ROWS_PER_TILE // 16, mk_zero, 0)

  pltpu.sync_copy(z_v, deg_sh.at[pl.ds(s * ROWS_PER_TILE, ROWS_PER_TILE)])
  plsc.subcore_barrier()

  def body(j, carry):
    pltpu.sync_copy(ones_v, deg_sh.at[d_v.at[j]], add=True)
    return carry
  lax.fori_loop(0, nb, body, 0)

  plsc.subcore_barrier()
  sl = pl.ds(s * ROWS_PER_TILE, ROWS_PER_TILE)
  pltpu.sync_copy(deg_sh.at[sl], out_hbm.at[c, sl])


def _sc_conv_body(y_hbm, sd_hbm, out_hbm, sd_v, sidx, didx, buf0, buf1,
                  acc_sh, g0, g1):
  c = lax.axis_index("c")
  s = lax.axis_index("s")
  w = c * NS + s
  nb = sd_v.shape[0]

  pltpu.sync_copy(sd_hbm.at[w], sd_v)

  def unpack(j, slot):
    for k in range(BATCH // 16):
      v = sd_v[j, pl.ds(k * 16, 16)]
      sidx[slot, pl.ds(k * 16, 16)] = v & 0x3FFF
      didx[slot, pl.ds(k * 16, 16)] = lax.shift_right_logical(v, 14)

  unpack(0, 0)
  pltpu.async_copy(y_hbm.at[sidx.at[0]], buf0, g0)

  def zrow(i, carry):
    def zcol(j, inner):
      buf1[i, pl.ds(j * 16, 16)] = jnp.zeros((16,), F32)
      return inner
    return lax.fori_loop(0, D // 16, zcol, carry)
  lax.fori_loop(0, BATCH, zrow, 0)
  for t in range(ROWS_PER_TILE // BATCH):
    pltpu.sync_copy(buf1, acc_sh.at[pl.ds(s * ROWS_PER_TILE + t * BATCH, BATCH)])

  unpack(1, 1)
  pltpu.async_copy(y_hbm.at[sidx.at[1]], buf1, g1)
  plsc.subcore_barrier()

  def body(i, carry):
    j0 = 2 * i
    j1 = 2 * i + 1
    pltpu.make_async_copy(y_hbm.at[sidx.at[0]], buf0, g0).wait()
    pltpu.sync_copy(buf0, acc_sh.at[didx.at[0]], add=True)

    @pl.when(j0 + 2 < nb)
    def _():
      unpack(j0 + 2, 0)
      pltpu.async_copy(y_hbm.at[sidx.at[0]], buf0, g0)

    pltpu.make_async_copy(y_hbm.at[sidx.at[1]], buf1, g1).wait()
    pltpu.sync_copy(buf1, acc_sh.at[didx.at[1]], add=True)

    @pl.when(j1 + 2 < nb)
    def _():
      unpack(j1 + 2, 1)
      pltpu.async_copy(y_hbm.at[sidx.at[1]], buf1, g1)

    return carry

  lax.fori_loop(0, nb // 2, body, 0)

  if nb % 2 == 1:
    pltpu.make_async_copy(y_hbm.at[sidx.at[0]], buf0, g0).wait()
    pltpu.sync_copy(buf0, acc_sh.at[didx.at[0]], add=True)

  plsc.subcore_barrier()
  sl = pl.ds(s * ROWS_PER_TILE, ROWS_PER_TILE)
  pltpu.sync_copy(acc_sh.at[sl], out_hbm.at[c, sl])


def _sc_deg(d_arr, nb):
  mesh = plsc.VectorSubcoreMesh(core_axis_name="c", subcore_axis_name="s")
  f = pl.kernel(
      _sc_deg_body,
      out_type=jax.ShapeDtypeStruct((NC, NPAD), F32),
      mesh=mesh,
      scratch_types=[
          pltpu.VMEM((nb, BATCH), jnp.int32),
          pltpu.VMEM((BATCH,), F32),
          pltpu.VMEM((ROWS_PER_TILE,), F32),
          pltpu.VMEM_SHARED((NPAD,), F32),
      ],
  )
  return f(d_arr)


def _sc_conv(y, sd_arr, nb):
  mesh = plsc.VectorSubcoreMesh(core_axis_name="c", subcore_axis_name="s")
  f = pl.kernel(
      _sc_conv_body,
      out_type=jax.ShapeDtypeStruct((NC, NPAD, D), F32),
      mesh=mesh,
      scratch_types=[
          pltpu.VMEM((nb, BATCH), jnp.int32),
          pltpu.VMEM((2, BATCH), jnp.int32),
          pltpu.VMEM((2, BATCH), jnp.int32),
          pltpu.VMEM((BATCH, D), F32),
          pltpu.VMEM((BATCH, D), F32),
          pltpu.VMEM_SHARED((NPAD, D), F32),
          pltpu.SemaphoreType.DMA,
          pltpu.SemaphoreType.DMA,
      ],
  )
  return f(y, sd_arr)


def _tc_mm_body(x_ref, w_ref, o_ref):
  o_ref[...] = jnp.dot(x_ref[...], w_ref[...], preferred_element_type=F32)


def _tc_mm(x_pad, w1):
  return pl.pallas_call(
      _tc_mm_body,
      grid=(GRID,),
      in_specs=[
          pl.BlockSpec((BM, D), lambda i: (i, 0)),
          pl.BlockSpec((D, D), lambda i: (0, 0)),
      ],
      out_specs=pl.BlockSpec((BM, D), lambda i: (i, 0)),
      out_shape=jax.ShapeDtypeStruct((NPAD, D), F32),
  )(x_pad, w1)


def _tc1_body(deg0_ref, deg1_ref, mask_ref, xw_ref, y_ref, dinv_ref):
  deg = deg0_ref[...] + deg1_ref[...] + 1.0
  dinv = lax.rsqrt(deg) * mask_ref[...]
  dinv_ref[...] = dinv
  y_ref[...] = xw_ref[...] * dinv


def _tc1(deg0, deg1, mask, xw):
  return pl.pallas_call(
      _tc1_body,
      grid=(GRID,),
      in_specs=[
          pl.BlockSpec((BM, 1), lambda i: (i, 0)),
          pl.BlockSpec((BM, 1), lambda i: (i, 0)),
          pl.BlockSpec((BM, 1), lambda i: (i, 0)),
          pl.BlockSpec((BM, D), lambda i: (i, 0)),
      ],
      out_specs=[
          pl.BlockSpec((BM, D), lambda i: (i, 0)),
          pl.BlockSpec((BM, 1), lambda i: (i, 0)),
      ],
      out_shape=[
          jax.ShapeDtypeStruct((NPAD, D), F32),
          jax.ShapeDtypeStruct((NPAD, 1), F32),
      ],
  )(deg0, deg1, mask, xw)


def _make_tc_boundary_body(with_mm):
  def body(a0_ref, a1_ref, y_ref, dinv_ref, mask_ref, b_ref, g_ref, be_ref,
           w_ref, o_ref, hbuf, st_ref):
    p = pl.program_id(0)
    i = pl.program_id(1)

    @pl.when(p == 0)
    def _():
      h = (dinv_ref[...] * (a0_ref[...] + a1_ref[...] + y_ref[...])
           + b_ref[...]) * mask_ref[...]
      hbuf[pl.ds(i * BM, BM), :] = h

      @pl.when(i == 0)
      def _():
        st_ref[...] = jnp.zeros_like(st_ref)

      s = jnp.sum(h, axis=0, keepdims=True)
      ss = jnp.sum(h * h, axis=0, keepdims=True)
      st_ref[...] += jnp.concatenate([s, ss], axis=0)

    @pl.when(p == 1)
    def _():
      st = st_ref[...]
      m = st[0:1, :] * (1.0 / N)
      v = st[1:2, :] * (1.0 / N) - m * m
      inv = lax.rsqrt(v + 1e-5) * g_ref[...]
      h = jnp.maximum((hbuf[pl.ds(i * BM, BM), :] - m) * inv + be_ref[...],
                      0.0)
      if with_mm:
        o_ref[...] = jnp.dot(h, w_ref[...],
                             preferred_element_type=F32) * dinv_ref[...]
      else:
        o_ref[...] = h * dinv_ref[...]

  return body


def _tc_boundary(a0, a1, y, dinv, mask, b, g, be, w, with_mm):
  hold = lambda p, i: (jnp.where(p == 0, i, GRID - 1), 0)
  phase1 = lambda p, i: (jnp.where(p == 0, 0, i), 0)
  both = lambda p, i: (i, 0)
  fixed = lambda p, i: (0, 0)
  return pl.pallas_call(
      _make_tc_boundary_body(with_mm),
      grid=(2, GRID),
      in_specs=[
          pl.BlockSpec((BM, D), hold),
          pl.BlockSpec((BM, D), hold),
          pl.BlockSpec((BM, D), hold),
          pl.BlockSpec((BM, 1), both),
          pl.BlockSpec((BM, 1), hold),
          pl.BlockSpec((1, D), fixed),
          pl.BlockSpec((1, D), fixed),
          pl.BlockSpec((1, D), fixed),
          pl.BlockSpec((D, D), fixed),
      ],
      out_specs=pl.BlockSpec((BM, D), phase1),
      out_shape=jax.ShapeDtypeStruct((NPAD, D), F32),
      scratch_shapes=[
          pltpu.VMEM((NPAD, D), F32),
          pltpu.VMEM((2, D), F32),
      ],
  )(a0, a1, y, dinv, mask, b, g, be, w)


def _tc_out_body(a0_ref, a1_ref, y_ref, dinv_ref, w_ref, b_ref, o_ref):
  z = dinv_ref[...] * (a0_ref[...] + a1_ref[...] + y_ref[...])
  o_ref[...] = jnp.dot(z, w_ref[...], preferred_element_type=F32) + b_ref[...]


def _tc_out(a0, a1, y, dinv, w3p, b3p):
  return pl.pallas_call(
      _tc_out_body,
      grid=(GRID,),
      in_specs=[
          pl.BlockSpec((BM, D), lambda i: (i, 0)),
          pl.BlockSpec((BM, D), lambda i: (i, 0)),
          pl.BlockSpec((BM, D), lambda i: (i, 0)),
          pl.BlockSpec((BM, 1), lambda i: (i, 0)),
          pl.BlockSpec((D, D), lambda i: (0, 0)),
          pl.BlockSpec((1, D), lambda i: (0, 0)),
      ],
      out_specs=pl.BlockSpec((BM, D), lambda i: (i, 0)),
      out_shape=jax.ShapeDtypeStruct((NPAD, D), F32),
  )(a0, a1, y, dinv, w3p, b3p)


def kernel(x, edge_index, W1, b1, g1, be1, W2, b2, g2, be2, W3, b3):
  n, d = x.shape
  e = edge_index.shape[1]
  c = W3.shape[1]

  epw = e // NW
  nb = -(-epw // BATCH)
  pad = nb * BATCH - epw
  src = edge_index[0].reshape(NW, epw)
  dst = edge_index[1].reshape(NW, epw)
  padv = (N + (jnp.arange(NW * pad, dtype=jnp.int32) % (NPAD - N))
          ).reshape(NW, pad)
  s_all = jnp.concatenate([src, padv], axis=1)
  d_all = jnp.concatenate([dst, padv], axis=1)
  d_arr = d_all.reshape(NW, nb, BATCH)
  sd_arr = (s_all | (d_all << 14)).reshape(NW, nb, BATCH)

  x_pad = jnp.pad(x, ((0, NPAD - n), (0, 0)))
  mask = (jnp.arange(NPAD, dtype=jnp.int32) < n).astype(F32).reshape(NPAD, 1)
  w3p = jnp.pad(W3, ((0, 0), (0, D - c)))
  b3p = jnp.pad(b3, (0, D - c)).reshape(1, D)
  b1r = b1.reshape(1, D)
  b2r = b2.reshape(1, D)
  g1r, be1r = g1.reshape(1, D), be1.reshape(1, D)
  g2r, be2r = g2.reshape(1, D), be2.reshape(1, D)

  degp = _sc_deg(d_arr, nb)
  xw1 = _tc_mm(x_pad, W1)
  deg0 = degp[0].reshape(NPAD, 1)
  deg1 = degp[1].reshape(NPAD, 1)

  y1, dinv = _tc1(deg0, deg1, mask, xw1)
  acc = _sc_conv(y1, sd_arr, nb)
  y2 = _tc_boundary(acc[0], acc[1], y1, dinv, mask, b1r, g1r, be1r, W2,
                    with_mm=True)

  acc = _sc_conv(y2, sd_arr, nb)
  y3 = _tc_boundary(acc[0], acc[1], y2, dinv, mask, b2r, g2r, be2r, W2,
                    with_mm=False)

  acc = _sc_conv(y3, sd_arr, nb)
  out = _tc_out(acc[0], acc[1], y3, dinv, w3p, b3p)

  return out[:n, :c]


# speedup vs baseline: 1.0036x; 1.0036x over previous
import jax
import jax.numpy as jnp
from jax import lax
from jax.experimental import pallas as pl
from jax.experimental.pallas import tpu as pltpu
from jax.experimental.pallas import tpu_sc as plsc

N = 10000
NPAD = 10240
D = 128
NC = 2
NS = 16
NW = NC * NS
ROWS_PER_TILE = NPAD // NS
BATCH = 128
BM = 2560
GRID = NPAD // BM
F32 = jnp.float32


def _sc_deg_body(d_hbm, out_hbm, d_v, ones_v, z_v, deg_sh):
  c = lax.axis_index("c")
  s = lax.axis_index("s")
  w = c * NS + s
  nb = d_v.shape[0]

  pltpu.sync_copy(d_hbm.at[w], d_v)

  def mk_ones(i, carry):
    ones_v[pl.ds(i * 16, 16)] = jnp.full((16,), 1.0, F32)
    return carry
  lax.fori_loop(0, BATCH // 16, mk_ones, 0)

  def mk_zero(i, carry):
    z_v[pl.ds(i * 16, 16)] = jnp.zeros((16,), F32)
    return carry
  lax.fori_loop(0, ROWS_PER_TILE // 16, mk_zero, 0)

  pltpu.sync_copy(z_v, deg_sh.at[pl.ds(s * ROWS_PER_TILE, ROWS_PER_TILE)])
  plsc.subcore_barrier()

  def body(j, carry):
    pltpu.sync_copy(ones_v, deg_sh.at[d_v.at[j]], add=True)
    return carry
  lax.fori_loop(0, nb, body, 0)

  plsc.subcore_barrier()
  sl = pl.ds(s * ROWS_PER_TILE, ROWS_PER_TILE)
  pltpu.sync_copy(deg_sh.at[sl], out_hbm.at[c, sl])


def _sc_conv_body(y_hbm, sd_hbm, out_hbm, sd_v, sidx, didx, buf0, buf1,
                  acc_sh, g0, g1):
  c = lax.axis_index("c")
  s = lax.axis_index("s")
  w = c * NS + s
  nb = sd_v.shape[0]

  pltpu.sync_copy(sd_hbm.at[w], sd_v)

  def unpack(j, slot):
    for k in range(BATCH // 16):
      v = sd_v[j, pl.ds(k * 16, 16)]
      sidx[slot, pl.ds(k * 16, 16)] = v & 0x3FFF
      didx[slot, pl.ds(k * 16, 16)] = lax.shift_right_logical(v, 14)

  unpack(0, 0)
  pltpu.async_copy(y_hbm.at[sidx.at[0]], buf0, g0)

  def zrow(i, carry):
    def zcol(j, inner):
      buf1[i, pl.ds(j * 16, 16)] = jnp.zeros((16,), F32)
      return inner
    return lax.fori_loop(0, D // 16, zcol, carry)
  lax.fori_loop(0, BATCH, zrow, 0)
  for t in range(ROWS_PER_TILE // BATCH):
    pltpu.sync_copy(buf1, acc_sh.at[pl.ds(s * ROWS_PER_TILE + t * BATCH, BATCH)])

  unpack(1, 1)
  pltpu.async_copy(y_hbm.at[sidx.at[1]], buf1, g1)
  plsc.subcore_barrier()

  def body(i, carry):
    j0 = 2 * i
    j1 = 2 * i + 1
    pltpu.make_async_copy(y_hbm.at[sidx.at[0]], buf0, g0).wait()
    pltpu.sync_copy(buf0, acc_sh.at[didx.at[0]], add=True)

    @pl.when(j0 + 2 < nb)
    def _():
      unpack(j0 + 2, 0)
      pltpu.async_copy(y_hbm.at[sidx.at[0]], buf0, g0)

    pltpu.make_async_copy(y_hbm.at[sidx.at[1]], buf1, g1).wait()
    pltpu.sync_copy(buf1, acc_sh.at[didx.at[1]], add=True)

    @pl.when(j1 + 2 < nb)
    def _():
      unpack(j1 + 2, 1)
      pltpu.async_copy(y_hbm.at[sidx.at[1]], buf1, g1)

    return carry

  lax.fori_loop(0, nb // 2, body, 0)

  if nb % 2 == 1:
    pltpu.make_async_copy(y_hbm.at[sidx.at[0]], buf0, g0).wait()
    pltpu.sync_copy(buf0, acc_sh.at[didx.at[0]], add=True)

  plsc.subcore_barrier()
  sl = pl.ds(s * ROWS_PER_TILE, ROWS_PER_TILE)
  pltpu.sync_copy(acc_sh.at[sl], out_hbm.at[c, sl])


def _sc_deg(d_arr, nb):
  mesh = plsc.VectorSubcoreMesh(core_axis_name="c", subcore_axis_name="s")
  f = pl.kernel(
      _sc_deg_body,
      out_type=jax.ShapeDtypeStruct((NC, NPAD), F32),
      mesh=mesh,
      scratch_types=[
          pltpu.VMEM((nb, BATCH), jnp.int32),
          pltpu.VMEM((BATCH,), F32),
          pltpu.VMEM((ROWS_PER_TILE,), F32),
          pltpu.VMEM_SHARED((NPAD,), F32),
      ],
  )
  return f(d_arr)


def _sc_conv(y, sd_arr, nb):
  mesh = plsc.VectorSubcoreMesh(core_axis_name="c", subcore_axis_name="s")
  f = pl.kernel(
      _sc_conv_body,
      out_type=jax.ShapeDtypeStruct((NC, NPAD, D), F32),
      mesh=mesh,
      scratch_types=[
          pltpu.VMEM((nb, BATCH), jnp.int32),
          pltpu.VMEM((2, BATCH), jnp.int32),
          pltpu.VMEM((2, BATCH), jnp.int32),
          pltpu.VMEM((BATCH, D), F32),
          pltpu.VMEM((BATCH, D), F32),
          pltpu.VMEM_SHARED((NPAD, D), F32),
          pltpu.SemaphoreType.DMA,
          pltpu.SemaphoreType.DMA,
      ],
  )
  return f(y, sd_arr)


def _tc1_body(deg0_ref, deg1_ref, mask_ref, x_ref, w_ref, y_ref, dinv_ref):
  deg = deg0_ref[...] + deg1_ref[...] + 1.0
  dinv = lax.rsqrt(deg) * mask_ref[...]
  dinv_ref[...] = dinv
  xw = jnp.dot(x_ref[...], w_ref[...], preferred_element_type=F32)
  y_ref[...] = xw * dinv


def _tc1(deg0, deg1, mask, x_pad, w1):
  return pl.pallas_call(
      _tc1_body,
      grid=(GRID,),
      in_specs=[
          pl.BlockSpec((BM, 1), lambda i: (i, 0)),
          pl.BlockSpec((BM, 1), lambda i: (i, 0)),
          pl.BlockSpec((BM, 1), lambda i: (i, 0)),
          pl.BlockSpec((BM, D), lambda i: (i, 0)),
          pl.BlockSpec((D, D), lambda i: (0, 0)),
      ],
      out_specs=[
          pl.BlockSpec((BM, D), lambda i: (i, 0)),
          pl.BlockSpec((BM, 1), lambda i: (i, 0)),
      ],
      out_shape=[
          jax.ShapeDtypeStruct((NPAD, D), F32),
          jax.ShapeDtypeStruct((NPAD, 1), F32),
      ],
  )(deg0, deg1, mask, x_pad, w1)


def _make_tc_boundary_body(with_mm):
  def body(a0_ref, a1_ref, y_ref, dinv_ref, mask_ref, b_ref, g_ref, be_ref,
           w_ref, o_ref, hbuf, st_ref):
    p = pl.program_id(0)
    i = pl.program_id(1)

    @pl.when(p == 0)
    def _():
      h = (dinv_ref[...] * (a0_ref[...] + a1_ref[...] + y_ref[...])
           + b_ref[...]) * mask_ref[...]
      hbuf[pl.ds(i * BM, BM), :] = h

      @pl.when(i == 0)
      def _():
        st_ref[...] = jnp.zeros_like(st_ref)

      s = jnp.sum(h, axis=0, keepdims=True)
      ss = jnp.sum(h * h, axis=0, keepdims=True)
      st_ref[...] += jnp.concatenate([s, ss], axis=0)

    @pl.when(p == 1)
    def _():
      st = st_ref[...]
      m = st[0:1, :] * (1.0 / N)
      v = st[1:2, :] * (1.0 / N) - m * m
      inv = lax.rsqrt(v + 1e-5) * g_ref[...]
      h = jnp.maximum((hbuf[pl.ds(i * BM, BM), :] - m) * inv + be_ref[...],
                      0.0)
      if with_mm:
        o_ref[...] = jnp.dot(h, w_ref[...],
                             preferred_element_type=F32) * dinv_ref[...]
      else:
        o_ref[...] = h * dinv_ref[...]

  return body


def _tc_boundary(a0, a1, y, dinv, mask, b, g, be, w, with_mm):
  hold = lambda p, i: (jnp.where(p == 0, i, GRID - 1), 0)
  phase1 = lambda p, i: (jnp.where(p == 0, 0, i), 0)
  both = lambda p, i: (i, 0)
  fixed = lambda p, i: (0, 0)
  return pl.pallas_call(
      _make_tc_boundary_body(with_mm),
      grid=(2, GRID),
      in_specs=[
          pl.BlockSpec((BM, D), hold),
          pl.BlockSpec((BM, D), hold),
          pl.BlockSpec((BM, D), hold),
          pl.BlockSpec((BM, 1), both),
          pl.BlockSpec((BM, 1), hold),
          pl.BlockSpec((1, D), fixed),
          pl.BlockSpec((1, D), fixed),
          pl.BlockSpec((1, D), fixed),
          pl.BlockSpec((D, D), fixed),
      ],
      out_specs=pl.BlockSpec((BM, D), phase1),
      out_shape=jax.ShapeDtypeStruct((NPAD, D), F32),
      scratch_shapes=[
          pltpu.VMEM((NPAD, D), F32),
          pltpu.VMEM((2, D), F32),
      ],
  )(a0, a1, y, dinv, mask, b, g, be, w)


def _tc_out_body(a0_ref, a1_ref, y_ref, dinv_ref, w_ref, b_ref, o_ref):
  z = dinv_ref[...] * (a0_ref[...] + a1_ref[...] + y_ref[...])
  o_ref[...] = jnp.dot(z, w_ref[...], preferred_element_type=F32) + b_ref[...]


def _tc_out(a0, a1, y, dinv, w3p, b3p):
  return pl.pallas_call(
      _tc_out_body,
      grid=(GRID,),
      in_specs=[
          pl.BlockSpec((BM, D), lambda i: (i, 0)),
          pl.BlockSpec((BM, D), lambda i: (i, 0)),
          pl.BlockSpec((BM, D), lambda i: (i, 0)),
          pl.BlockSpec((BM, 1), lambda i: (i, 0)),
          pl.BlockSpec((D, D), lambda i: (0, 0)),
          pl.BlockSpec((1, D), lambda i: (0, 0)),
      ],
      out_specs=pl.BlockSpec((BM, D), lambda i: (i, 0)),
      out_shape=jax.ShapeDtypeStruct((NPAD, D), F32),
  )(a0, a1, y, dinv, w3p, b3p)


def kernel(x, edge_index, W1, b1, g1, be1, W2, b2, g2, be2, W3, b3):
  n, d = x.shape
  e = edge_index.shape[1]
  c = W3.shape[1]

  epw = e // NW
  nb = -(-epw // BATCH)
  pad = nb * BATCH - epw
  src = edge_index[0].reshape(NW, epw)
  dst = edge_index[1].reshape(NW, epw)
  padv = (N + (jnp.arange(NW * pad, dtype=jnp.int32) % (NPAD - N))
          ).reshape(NW, pad)
  s_all = jnp.concatenate([src, padv], axis=1)
  d_all = jnp.concatenate([dst, padv], axis=1)
  d_arr = d_all.reshape(NW, nb, BATCH)
  sd_arr = (s_all | (d_all << 14)).reshape(NW, nb, BATCH)

  x_pad = jnp.pad(x, ((0, NPAD - n), (0, 0)))
  mask = (jnp.arange(NPAD, dtype=jnp.int32) < n).astype(F32).reshape(NPAD, 1)
  w3p = jnp.pad(W3, ((0, 0), (0, D - c)))
  b3p = jnp.pad(b3, (0, D - c)).reshape(1, D)
  b1r = b1.reshape(1, D)
  b2r = b2.reshape(1, D)
  g1r, be1r = g1.reshape(1, D), be1.reshape(1, D)
  g2r, be2r = g2.reshape(1, D), be2.reshape(1, D)

  degp = _sc_deg(d_arr, nb)
  deg0 = degp[0].reshape(NPAD, 1)
  deg1 = degp[1].reshape(NPAD, 1)

  y1, dinv = _tc1(deg0, deg1, mask, x_pad, W1)
  acc = _sc_conv(y1, sd_arr, nb)
  y2 = _tc_boundary(acc[0], acc[1], y1, dinv, mask, b1r, g1r, be1r, W2,
                    with_mm=True)

  acc = _sc_conv(y2, sd_arr, nb)
  y3 = _tc_boundary(acc[0], acc[1], y2, dinv, mask, b2r, g2r, be2r, W2,
                    with_mm=False)

  acc = _sc_conv(y3, sd_arr, nb)
  out = _tc_out(acc[0], acc[1], y3, dinv, w3p, b3p)

  return out[:n, :c]
